# Initial kernel scaffold; baseline (speedup 1.0000x reference)
import jax
import jax.numpy as jnp
from jax.experimental import pallas as pl


def kernel(x, W_gate, W_noise):
    raise NotImplementedError("write your pallas kernel here")


# speedup vs baseline: 1.9332x; 1.9332x over previous
import jax
import jax.numpy as jnp
from jax.experimental import pallas as pl
from jax.experimental.pallas import tpu as pltpu

BLOCK_R = 2048
NUM_EXPERTS = 64
MODEL_DIM = 768


def _gating_body(x_ref, wt_ref, logits_ref, w_ref, i_ref):
    x_blk = x_ref[...]
    logits = jnp.dot(x_blk, wt_ref[...], preferred_element_type=jnp.float32)
    logits_ref[...] = logits

    iota = jax.lax.broadcasted_iota(jnp.int32, logits.shape, 1)
    m1 = jnp.max(logits, axis=1, keepdims=True)
    i1 = jnp.min(jnp.where(logits == m1, iota, NUM_EXPERTS), axis=1,
                 keepdims=True)
    masked = jnp.where(iota == i1, -jnp.inf, logits)
    m2 = jnp.max(masked, axis=1, keepdims=True)
    i2 = jnp.min(jnp.where(masked == m2, iota, NUM_EXPERTS), axis=1,
                 keepdims=True)

    s = jnp.exp(m2 - m1)
    denom = 1.0 + s
    w1 = 1.0 / denom
    w2 = s / denom

    lane2 = jax.lax.broadcasted_iota(jnp.int32, (logits.shape[0], 2), 1)
    w_ref[...] = jnp.where(lane2 == 0, w1, w2)
    i_ref[...] = jnp.where(lane2 == 0, i1, i2)


def kernel(x, W_gate, W_noise):
    del W_noise
    n = x.shape[0]
    wt = W_gate.T

    grid = (n // BLOCK_R,)
    logits, weights, indices = pl.pallas_call(
        _gating_body,
        grid=grid,
        in_specs=[
            pl.BlockSpec((BLOCK_R, MODEL_DIM), lambda i: (i, 0)),
            pl.BlockSpec((MODEL_DIM, NUM_EXPERTS), lambda i: (0, 0)),
        ],
        out_specs=[
            pl.BlockSpec((BLOCK_R, NUM_EXPERTS), lambda i: (i, 0)),
            pl.BlockSpec((BLOCK_R, 2), lambda i: (i, 0)),
            pl.BlockSpec((BLOCK_R, 2), lambda i: (i, 0)),
        ],
        out_shape=[
            jax.ShapeDtypeStruct((n, NUM_EXPERTS), jnp.float32),
            jax.ShapeDtypeStruct((n, 2), jnp.float32),
            jax.ShapeDtypeStruct((n, 2), jnp.int32),
        ],
    )(x, wt)
    return weights, indices, logits


# speedup vs baseline: 2.0308x; 1.0505x over previous
import jax
import jax.numpy as jnp
from jax.experimental import pallas as pl
from jax.experimental.pallas import tpu as pltpu

BLOCK_R = 4096
NUM_EXPERTS = 64
MODEL_DIM = 768


def _gating_body(x_ref, wt_ref, logits_ref, w_ref, i_ref):
    x_blk = x_ref[...]
    logits = jnp.dot(x_blk, wt_ref[...], preferred_element_type=jnp.float32)
    logits_ref[...] = logits

    iota = jax.lax.broadcasted_iota(jnp.int32, logits.shape, 1)
    m1 = jnp.max(logits, axis=1, keepdims=True)
    i1 = jnp.min(jnp.where(logits == m1, iota, NUM_EXPERTS), axis=1,
                 keepdims=True)
    masked = jnp.where(iota == i1, -jnp.inf, logits)
    m2 = jnp.max(masked, axis=1, keepdims=True)
    i2 = jnp.min(jnp.where(masked == m2, iota, NUM_EXPERTS), axis=1,
                 keepdims=True)

    s = jnp.exp(m2 - m1)
    denom = 1.0 + s
    w1 = 1.0 / denom
    w2 = s / denom

    lane2 = jax.lax.broadcasted_iota(jnp.int32, (logits.shape[0], 2), 1)
    w_ref[...] = jnp.where(lane2 == 0, w1, w2)
    i_ref[...] = jnp.where(lane2 == 0, i1, i2)


def kernel(x, W_gate, W_noise):
    del W_noise
    n = x.shape[0]
    wt = W_gate.T

    grid = (n // BLOCK_R,)
    logits, weights, indices = pl.pallas_call(
        _gating_body,
        grid=grid,
        in_specs=[
            pl.BlockSpec((BLOCK_R, MODEL_DIM), lambda i: (i, 0)),
            pl.BlockSpec((MODEL_DIM, NUM_EXPERTS), lambda i: (0, 0)),
        ],
        out_specs=[
            pl.BlockSpec((BLOCK_R, NUM_EXPERTS), lambda i: (i, 0)),
            pl.BlockSpec((BLOCK_R, 2), lambda i: (i, 0)),
            pl.BlockSpec((BLOCK_R, 2), lambda i: (i, 0)),
        ],
        out_shape=[
            jax.ShapeDtypeStruct((n, NUM_EXPERTS), jnp.float32),
            jax.ShapeDtypeStruct((n, 2), jnp.float32),
            jax.ShapeDtypeStruct((n, 2), jnp.int32),
        ],
    )(x, wt)
    return weights, indices, logits
